# initial kernel scaffold (unmeasured)
import jax
import jax.numpy as jnp
from jax import lax
from jax.experimental import pallas as pl
from jax.experimental.pallas import tpu as pltpu

N_DEV = 8


def kernel(x, w_mat, scale_x, scale_w):
    m_per, k = x.shape
    n = w_mat.shape[1]

    def body(x_ref, w_ref, sx_ref, sw_ref, out_ref, comm_ref, send_sems, recv_sems):
        my = lax.axis_index("i")
        left = lax.rem(my - 1 + N_DEV, N_DEV)
        right = lax.rem(my + 1, N_DEV)

        barrier_sem = pltpu.get_barrier_semaphore()
        for nbr in [left, right]:
            pl.semaphore_signal(
                barrier_sem, inc=1,
                device_id=(nbr,), device_id_type=pl.DeviceIdType.MESH,
            )
        pl.semaphore_wait(barrier_sem, 2)

        scale = sx_ref[0] * sw_ref[0]

        def compute(origin, chunk):
            acc = lax.dot_general(
                chunk, w_ref[:, :],
                (((1,), (0,)), ((), ())),
                preferred_element_type=jnp.float32,
            )
            y = acc * scale
            out_ref[pl.ds(origin * m_per, m_per), :] = y * jax.nn.sigmoid(y)

        comm_ref[0, :, :] = x_ref[:, :]
        compute(my, x_ref[:, :])

        for h in range(N_DEV - 1):
            rdma = pltpu.make_async_remote_copy(
                src_ref=comm_ref.at[h],
                dst_ref=comm_ref.at[h + 1],
                send_sem=send_sems.at[h],
                recv_sem=recv_sems.at[h + 1],
                device_id=(right,),
                device_id_type=pl.DeviceIdType.MESH,
            )
            rdma.start()
            rdma.wait()

            origin = lax.rem(my - (h + 1) + N_DEV, N_DEV)
            compute(origin, comm_ref[h + 1, :, :])

    return pl.pallas_call(
        body,
        out_shape=jax.ShapeDtypeStruct((N_DEV * m_per, n), jnp.float32),
        in_specs=[
            pl.BlockSpec(memory_space=pltpu.VMEM),
            pl.BlockSpec(memory_space=pltpu.VMEM),
            pl.BlockSpec(memory_space=pltpu.SMEM),
            pl.BlockSpec(memory_space=pltpu.SMEM),
        ],
        out_specs=pl.BlockSpec(memory_space=pltpu.VMEM),
        scratch_shapes=[
            pltpu.VMEM((N_DEV, m_per, k), x.dtype),
            pltpu.SemaphoreType.DMA((N_DEV,)),
            pltpu.SemaphoreType.DMA((N_DEV,)),
        ],
        compiler_params=pltpu.CompilerParams(collective_id=0),
    )(x, w_mat, scale_x, scale_w)


# baseline (device time: 191837 ns/iter reference)
import jax
import jax.numpy as jnp
from jax import lax
from jax.experimental import pallas as pl
from jax.experimental.pallas import tpu as pltpu

N_DEV = 8


def kernel(x, w_mat, scale_x, scale_w):
    m_per, k = x.shape
    n = w_mat.shape[1]

    def body(x_ref, w_ref, sx_ref, sw_ref, out_ref, comm_ref, w8_ref, send_sems, recv_sems):
        my = lax.axis_index("i")
        left = lax.rem(my - 1 + N_DEV, N_DEV)
        right = lax.rem(my + 1, N_DEV)

        barrier_sem = pltpu.get_barrier_semaphore()
        for nbr in [left, right]:
            pl.semaphore_signal(
                barrier_sem, inc=1,
                device_id=(nbr,), device_id_type=pl.DeviceIdType.MESH,
            )
        pl.semaphore_wait(barrier_sem, 2)

        scale = sx_ref[0] * sw_ref[0]

        w8_ref[:, :] = w_ref[:, :].astype(jnp.float8_e5m2)

        def compute(origin, chunk):
            acc = lax.dot_general(
                chunk, w8_ref[:, :],
                (((1,), (0,)), ((), ())),
                preferred_element_type=jnp.float32,
            )
            y = acc * scale
            out_ref[pl.ds(origin * m_per, m_per), :] = y * jax.nn.sigmoid(y)

        comm_ref[0, :, :] = x_ref[:, :].astype(jnp.float8_e5m2)
        compute(my, comm_ref[0, :, :])

        for h in range(N_DEV - 1):
            rdma = pltpu.make_async_remote_copy(
                src_ref=comm_ref.at[h],
                dst_ref=comm_ref.at[h + 1],
                send_sem=send_sems.at[h],
                recv_sem=recv_sems.at[h + 1],
                device_id=(right,),
                device_id_type=pl.DeviceIdType.MESH,
            )
            rdma.start()
            rdma.wait()

            origin = lax.rem(my - (h + 1) + N_DEV, N_DEV)
            compute(origin, comm_ref[h + 1, :, :])

    return pl.pallas_call(
        body,
        out_shape=jax.ShapeDtypeStruct((N_DEV * m_per, n), jnp.float32),
        in_specs=[
            pl.BlockSpec(memory_space=pltpu.VMEM),
            pl.BlockSpec(memory_space=pltpu.VMEM),
            pl.BlockSpec(memory_space=pltpu.SMEM),
            pl.BlockSpec(memory_space=pltpu.SMEM),
        ],
        out_specs=pl.BlockSpec(memory_space=pltpu.VMEM),
        scratch_shapes=[
            pltpu.VMEM((N_DEV, m_per, k), jnp.float8_e5m2),
            pltpu.VMEM((k, n), jnp.float8_e5m2),
            pltpu.SemaphoreType.DMA((N_DEV,)),
            pltpu.SemaphoreType.DMA((N_DEV,)),
        ],
        compiler_params=pltpu.CompilerParams(collective_id=0),
    )(x, w_mat, scale_x, scale_w)


# device time: 100988 ns/iter; 1.8996x vs baseline; 1.8996x over previous
import jax
import jax.numpy as jnp
from jax import lax
from jax.experimental import pallas as pl
from jax.experimental.pallas import tpu as pltpu

N_DEV = 8
LEVELS = N_DEV // 2


def kernel(x, w_mat, scale_x, scale_w):
    m_per, k = x.shape
    n = w_mat.shape[1]
    half = m_per // 2

    def body(x_ref, w_ref, sx_ref, sw_ref, out_ref,
             own_ref, r_ref, l_ref, w8_ref,
             r_send, r_recv, l_send, l_recv):
        my = lax.axis_index("i")
        left = lax.rem(my - 1 + N_DEV, N_DEV)
        right = lax.rem(my + 1, N_DEV)

        barrier_sem = pltpu.get_barrier_semaphore()
        for nbr in [left, right]:
            pl.semaphore_signal(
                barrier_sem, inc=1,
                device_id=(nbr,), device_id_type=pl.DeviceIdType.MESH,
            )
        pl.semaphore_wait(barrier_sem, 2)

        scale = sx_ref[0] * sw_ref[0]
        own_ref[:, :] = x_ref[:, :].astype(jnp.float8_e5m2)
        w8_ref[:, :] = w_ref[:, :].astype(jnp.float8_e5m2)

        def gemm(origin, row_off, chunk):
            acc = lax.dot_general(
                chunk, w8_ref[:, :],
                (((1,), (0,)), ((), ())),
                preferred_element_type=jnp.float32,
            )
            y = acc * scale
            rows = chunk.shape[0]
            out_ref[pl.ds(origin * m_per + row_off, rows), :] = (
                y * jax.nn.sigmoid(y))

        def make(lvl, stream_ref, send_sems, recv_sems, dst_dev):
            if lvl == 0:
                src = own_ref
            else:
                src = stream_ref.at[lvl - 1]
            dst = stream_ref.at[lvl]
            if lvl == LEVELS - 1:
                rs = pl.ds(0, half) if dst_dev is right else pl.ds(half, half)
                src = src.at[rs, :]
                dst = dst.at[rs, :]
            return pltpu.make_async_remote_copy(
                src_ref=src, dst_ref=dst,
                send_sem=send_sems.at[lvl], recv_sem=recv_sems.at[lvl],
                device_id=(dst_dev,), device_id_type=pl.DeviceIdType.MESH,
            )

        r_rdmas = [None] * LEVELS
        l_rdmas = [None] * LEVELS
        r_rdmas[0] = make(0, r_ref, r_send, r_recv, right)
        l_rdmas[0] = make(0, l_ref, l_send, l_recv, left)
        r_rdmas[0].start()
        l_rdmas[0].start()

        gemm(my, 0, own_ref[:, :])

        for lvl in range(LEVELS):
            r_rdmas[lvl].wait_recv()
            l_rdmas[lvl].wait_recv()
            if lvl + 1 < LEVELS:
                r_rdmas[lvl + 1] = make(lvl + 1, r_ref, r_send, r_recv, right)
                l_rdmas[lvl + 1] = make(lvl + 1, l_ref, l_send, l_recv, left)
                r_rdmas[lvl + 1].start()
                l_rdmas[lvl + 1].start()
            origin_r = lax.rem(my - (lvl + 1) + N_DEV, N_DEV)
            origin_l = lax.rem(my + (lvl + 1), N_DEV)
            if lvl < LEVELS - 1:
                gemm(origin_r, 0, r_ref[lvl, :, :])
                gemm(origin_l, 0, l_ref[lvl, :, :])
            else:
                gemm(origin_r, 0, r_ref[lvl, pl.ds(0, half), :])
                gemm(origin_l, half, l_ref[lvl, pl.ds(half, half), :])

        for d in r_rdmas + l_rdmas:
            d.wait_send()

    return pl.pallas_call(
        body,
        out_shape=jax.ShapeDtypeStruct((N_DEV * m_per, n), jnp.float32),
        in_specs=[
            pl.BlockSpec(memory_space=pltpu.VMEM),
            pl.BlockSpec(memory_space=pltpu.VMEM),
            pl.BlockSpec(memory_space=pltpu.SMEM),
            pl.BlockSpec(memory_space=pltpu.SMEM),
        ],
        out_specs=pl.BlockSpec(memory_space=pltpu.VMEM),
        scratch_shapes=[
            pltpu.VMEM((m_per, k), jnp.float8_e5m2),
            pltpu.VMEM((LEVELS, m_per, k), jnp.float8_e5m2),
            pltpu.VMEM((LEVELS, m_per, k), jnp.float8_e5m2),
            pltpu.VMEM((k, n), jnp.float8_e5m2),
            pltpu.SemaphoreType.DMA((LEVELS,)),
            pltpu.SemaphoreType.DMA((LEVELS,)),
            pltpu.SemaphoreType.DMA((LEVELS,)),
            pltpu.SemaphoreType.DMA((LEVELS,)),
        ],
        compiler_params=pltpu.CompilerParams(collective_id=0),
    )(x, w_mat, scale_x, scale_w)


# device time: 94051 ns/iter; 2.0397x vs baseline; 1.0738x over previous
import jax
import jax.numpy as jnp
from jax import lax
from jax.experimental import pallas as pl
from jax.experimental.pallas import tpu as pltpu

N_DEV = 8
LEVELS = N_DEV // 2


def kernel(x, w_mat, scale_x, scale_w):
    m_per, k = x.shape
    n = w_mat.shape[1]
    half = m_per // 2

    def active(stream, lane, lvl):
        long_lane = 0 if stream == 0 else 1
        return lvl < LEVELS - 1 or lane == long_lane

    def body(x_ref, w_ref, sx_ref, sw_ref, out_ref,
             own_ref, r_ref, l_ref, w8_ref,
             r_send, r_recv, l_send, l_recv):
        my = lax.axis_index("i")
        left = lax.rem(my - 1 + N_DEV, N_DEV)
        right = lax.rem(my + 1, N_DEV)

        barrier_sem = pltpu.get_barrier_semaphore()
        for nbr in [left, right]:
            pl.semaphore_signal(
                barrier_sem, inc=1,
                device_id=(nbr,), device_id_type=pl.DeviceIdType.MESH,
            )
        pl.semaphore_wait(barrier_sem, 2)

        scale = sx_ref[0] * sw_ref[0]

        streams = (
            (r_ref, r_send, r_recv, right),
            (l_ref, l_send, l_recv, left),
        )

        def make(stream, lane, lvl):
            buf, send_sems, recv_sems, dst_dev = streams[stream]
            rs = pl.ds(lane * half, half)
            if lvl == 0:
                src = own_ref.at[rs, :]
            else:
                src = buf.at[lvl - 1, rs, :]
            return pltpu.make_async_remote_copy(
                src_ref=src,
                dst_ref=buf.at[lvl, rs, :],
                send_sem=send_sems.at[lvl, lane],
                recv_sem=recv_sems.at[lvl, lane],
                device_id=(dst_dev,), device_id_type=pl.DeviceIdType.MESH,
            )

        def gemm(origin, row_off, chunk):
            acc = lax.dot_general(
                chunk, w8_ref[:, :],
                (((1,), (0,)), ((), ())),
                preferred_element_type=jnp.float32,
            )
            y = acc * scale
            out_ref[pl.ds(origin * m_per + row_off, chunk.shape[0]), :] = (
                y * jax.nn.sigmoid(y))

        rdmas = {}
        for lane in (0, 1):
            rs = pl.ds(lane * half, half)
            own_ref[rs, :] = x_ref[rs, :].astype(jnp.float8_e5m2)
            for stream in (0, 1):
                d = make(stream, lane, 0)
                d.start()
                rdmas[(stream, lane, 0)] = d

        w8_ref[:, :] = w_ref[:, :].astype(jnp.float8_e5m2)
        gemm(my, 0, own_ref[:, :])

        for lvl in range(LEVELS):
            order = [(0, 0), (1, 1), (0, 1), (1, 0)]
            arrived = []
            for stream, lane in order:
                if not active(stream, lane, lvl):
                    continue
                rdmas[(stream, lane, lvl)].wait_recv()
                if active(stream, lane, lvl + 1) and lvl + 1 < LEVELS:
                    d = make(stream, lane, lvl + 1)
                    d.start()
                    rdmas[(stream, lane, lvl + 1)] = d
                arrived.append((stream, lane))
            for stream, lane in arrived:
                buf = streams[stream][0]
                if stream == 0:
                    origin = lax.rem(my - (lvl + 1) + N_DEV, N_DEV)
                else:
                    origin = lax.rem(my + (lvl + 1), N_DEV)
                gemm(origin, lane * half, buf[lvl, pl.ds(lane * half, half), :])

        for d in rdmas.values():
            d.wait_send()

    return pl.pallas_call(
        body,
        out_shape=jax.ShapeDtypeStruct((N_DEV * m_per, n), jnp.float32),
        in_specs=[
            pl.BlockSpec(memory_space=pltpu.VMEM),
            pl.BlockSpec(memory_space=pltpu.VMEM),
            pl.BlockSpec(memory_space=pltpu.SMEM),
            pl.BlockSpec(memory_space=pltpu.SMEM),
        ],
        out_specs=pl.BlockSpec(memory_space=pltpu.VMEM),
        scratch_shapes=[
            pltpu.VMEM((m_per, k), jnp.float8_e5m2),
            pltpu.VMEM((LEVELS, m_per, k), jnp.float8_e5m2),
            pltpu.VMEM((LEVELS, m_per, k), jnp.float8_e5m2),
            pltpu.VMEM((k, n), jnp.float8_e5m2),
            pltpu.SemaphoreType.DMA((LEVELS, 2)),
            pltpu.SemaphoreType.DMA((LEVELS, 2)),
            pltpu.SemaphoreType.DMA((LEVELS, 2)),
            pltpu.SemaphoreType.DMA((LEVELS, 2)),
        ],
        compiler_params=pltpu.CompilerParams(collective_id=0),
    )(x, w_mat, scale_x, scale_w)
